# baseline (speedup 1.0000x reference)
import jax
import jax.numpy as jnp
from jax import lax
from jax.experimental import pallas as pl
from jax.experimental.pallas import tpu as pltpu
from jax.experimental.pallas import tpu_sc as plsc

N = 10000
E = 320000
NEG = 0.2
RW = 80
CB = 80
NSLOT = 5
TILES = 32
EPT = E // TILES
CHUNKS = EPT // CB
NGROUP = CHUNKS // NSLOT
RPT = 632
RPT_LAST = N - 15 * RPT


def _attn_tables(act, Wg_ref, asv_ref, adv_ref, hext_ref, stab_ref):
    g = jnp.dot(act, Wg_ref[...], preferred_element_type=jnp.float32)
    a_s = lax.dot_general(asv_ref[...], g, (((1,), (1,)), ((), ())),
                          preferred_element_type=jnp.float32)
    a_d = lax.dot_general(adv_ref[...], g, (((1,), (1,)), ((), ())),
                          preferred_element_type=jnp.float32)
    s = a_s + a_d
    c = jnp.where(s >= 0, s, NEG * s)
    stab_ref[...] = jnp.concatenate(
        [a_s, a_d, c, jnp.zeros((5, N), jnp.float32)], axis=0)
    hext_ref[...] = jnp.concatenate(
        [g, jnp.ones((N, 1), jnp.float32),
         jnp.zeros((N, RW - 65), jnp.float32)], axis=1)


def _prep0_body(x_ref, Wn_ref, bn_ref, Wg_ref, asv_ref, adv_ref,
                hext_ref, stab_ref):
    act = jnp.dot(x_ref[...], Wn_ref[...],
                  preferred_element_type=jnp.float32) + bn_ref[...]
    _attn_tables(act, Wg_ref, asv_ref, adv_ref, hext_ref, stab_ref)


def _gat_out(acc_ref, hextp_ref, bg_ref):
    ssum = acc_ref[0] + acc_ref[1] - hextp_ref[...]
    return ssum[:, :64] / (ssum[:, 64:65] + 1e-16) + bg_ref[...]


def _prep_mid_body(acc_ref, hextp_ref, bg_ref,
                   Wg_ref, asv_ref, adv_ref, hext_ref, stab_ref):
    act = jnp.maximum(_gat_out(acc_ref, hextp_ref, bg_ref), 0.0)
    _attn_tables(act, Wg_ref, asv_ref, adv_ref, hext_ref, stab_ref)


def _head_body(acc_ref, hextp_ref, bg_ref,
               Wp1_ref, bp1_ref, Wp2_ref, bp2_ref,
               Wc1_ref, bc1_ref, Wc2_ref, bc2_ref,
               Wt1_ref, bt1_ref, Wt2_ref, bt2_ref,
               scores_ref, types_ref, ge_ref):
    h = _gat_out(acc_ref, hextp_ref, bg_ref)
    gm = jnp.mean(h, axis=0, keepdims=True)
    gx = jnp.max(h, axis=0, keepdims=True)
    gr = jnp.concatenate([gm, gx], axis=1)
    ge = jnp.maximum(gr @ Wp1_ref[...] + bp1_ref[...], 0.0) @ Wp2_ref[...] \
        + bp2_ref[...]
    scores_ref[...] = jax.nn.sigmoid(
        jnp.maximum(ge @ Wc1_ref[...] + bc1_ref[...], 0.0) @ Wc2_ref[...]
        + bc2_ref[...])
    types_ref[...] = jnp.maximum(ge @ Wt1_ref[...] + bt1_ref[...], 0.0) \
        @ Wt2_ref[...] + bt2_ref[...]
    ge_ref[...] = ge


def _full16(v):
    return jnp.full((16,), v, jnp.int32)


def _sc_edge(src_hbm, dst_hbm, hext_hbm, as_hbm, ad_hbm, c_hbm, acc_hbm,
             tabA, tabB, tabC, srcidx, dstidx, rows, acc_sh,
             ids_sem, gat_sem, scr_sem):
    cid = lax.axis_index("c")
    sid = lax.axis_index("s")
    wid = cid * 16 + sid
    pltpu.sync_copy(as_hbm, tabA)
    pltpu.sync_copy(ad_hbm, tabB)
    pltpu.sync_copy(c_hbm, tabC)
    r0 = sid * RPT

    @pl.when(sid < 15)
    def _():
        pltpu.sync_copy(hext_hbm.at[pl.ds(r0, RPT)], acc_sh.at[pl.ds(r0, RPT)])

    @pl.when(sid == 15)
    def _():
        pltpu.sync_copy(hext_hbm.at[pl.ds(15 * RPT, RPT_LAST)],
                        acc_sh.at[pl.ds(15 * RPT, RPT_LAST)])

    plsc.subcore_barrier()

    ebase = wid * EPT

    def _fire_ids(k, s):
        base = ebase + k * CB
        pltpu.async_copy(src_hbm.at[pl.ds(base, CB)], srcidx.at[s],
                         ids_sem.at[s])
        pltpu.async_copy(dst_hbm.at[pl.ds(base, CB)], dstidx.at[s],
                         ids_sem.at[s])

    def _wait_ids(k, s):
        base = ebase + k * CB
        pltpu.make_async_copy(src_hbm.at[pl.ds(base, CB)], srcidx.at[s],
                              ids_sem.at[s]).wait()
        pltpu.make_async_copy(dst_hbm.at[pl.ds(base, CB)], dstidx.at[s],
                              ids_sem.at[s]).wait()

    def _fire_gather(s):
        pltpu.async_copy(hext_hbm.at[srcidx.at[s]], rows.at[s],
                         gat_sem.at[s])

    def _wait_gather(s):
        pltpu.make_async_copy(hext_hbm.at[srcidx.at[s]], rows.at[s],
                              gat_sem.at[s]).wait()

    def _fire_scat(s):
        pltpu.async_copy(rows.at[s], acc_sh.at[dstidx.at[s]],
                         scr_sem.at[s], add=True)

    def _wait_scat(s):
        pltpu.make_async_copy(rows.at[s], acc_sh.at[dstidx.at[s]],
                              scr_sem.at[s]).wait()

    for s in range(3):
        _fire_ids(s, s)
    for s in range(2):
        _wait_ids(s, s)
        _fire_gather(s)

    def group(gi, carry):
        for s in range(NSLOT):
            k = gi * NSLOT + s
            _wait_gather(s)
            for g5 in range(CB // 16):
                sl = pl.ds(g5 * 16, 16)
                sidx = srcidx[s, sl]
                didx = dstidx[s, sl]
                a_s = plsc.load_gather(tabA, [sidx])
                a_d = plsc.load_gather(tabB, [didx])
                cc = plsc.load_gather(tabC, [didx])
                t = a_s + a_d
                e = jnp.where(t >= 0, t, NEG * t)
                w = jnp.exp(e - cc)
                for l in range(16):
                    i = g5 * 16 + l
                    wl = w.at[_full16(l)].get(mode="promise_in_bounds")
                    for jb in range(RW // 16):
                        slj = pl.ds(jb * 16, 16)
                        rows[s, i, slj] = rows[s, i, slj] * wl
            _fire_scat(s)

            s3 = (s + 3) % NSLOT

            @pl.when(k < CHUNKS - 3)
            def _():
                @pl.when(k >= 2)
                def _():
                    _wait_scat(s3)

                _fire_ids(k + 3, s3)

            s2 = (s + 2) % NSLOT

            @pl.when(k < CHUNKS - 2)
            def _():
                _wait_ids(k + 2, s2)
                _fire_gather(s2)

        return carry

    lax.fori_loop(0, NGROUP, group, 0)
    for s in range(NSLOT):
        _wait_scat(s)
    plsc.subcore_barrier()

    @pl.when(sid < 15)
    def _():
        pltpu.sync_copy(acc_sh.at[pl.ds(r0, RPT)],
                        acc_hbm.at[cid, pl.ds(r0, RPT)])

    @pl.when(sid == 15)
    def _():
        pltpu.sync_copy(acc_sh.at[pl.ds(15 * RPT, RPT_LAST)],
                        acc_hbm.at[cid, pl.ds(15 * RPT, RPT_LAST)])


_sc_call = pl.kernel(
    _sc_edge,
    mesh=plsc.VectorSubcoreMesh(core_axis_name="c", subcore_axis_name="s"),
    out_type=jax.ShapeDtypeStruct((2, N, RW), jnp.float32),
    scratch_types=[
        pltpu.VMEM((N,), jnp.float32),
        pltpu.VMEM((N,), jnp.float32),
        pltpu.VMEM((N,), jnp.float32),
        pltpu.VMEM((NSLOT, CB), jnp.int32),
        pltpu.VMEM((NSLOT, CB), jnp.int32),
        pltpu.VMEM((NSLOT, CB, RW), jnp.float32),
        pltpu.VMEM_SHARED((N, RW), jnp.float32),
        pltpu.SemaphoreType.DMA((NSLOT,)),
        pltpu.SemaphoreType.DMA((NSLOT,)),
        pltpu.SemaphoreType.DMA((NSLOT,)),
    ],
    compiler_params=pltpu.CompilerParams(needs_layout_passes=False,
                                         use_tc_tiling_on_sc=False),
)

_prep0 = pl.pallas_call(
    _prep0_body,
    out_shape=(jax.ShapeDtypeStruct((N, RW), jnp.float32),
               jax.ShapeDtypeStruct((8, N), jnp.float32)),
)

_prep_mid = pl.pallas_call(
    _prep_mid_body,
    out_shape=(jax.ShapeDtypeStruct((N, RW), jnp.float32),
               jax.ShapeDtypeStruct((8, N), jnp.float32)),
)

_head = pl.pallas_call(
    _head_body,
    out_shape=(jax.ShapeDtypeStruct((1, 1), jnp.float32),
               jax.ShapeDtypeStruct((1, 6), jnp.float32),
               jax.ShapeDtypeStruct((1, 32), jnp.float32)),
)


def kernel(x, edge_index, edge_attr, Wn, bn, Wg0, as0, ad0, bg0,
           Wg1, as1, ad1, bg1, Wg2, as2, ad2, bg2,
           Wp1, bp1, Wp2, bp2, Wc1, bc1, Wc2, bc2, Wt1, bt1, Wt2, bt2):
    src = edge_index[0]
    dst = edge_index[1]
    asv = [a.reshape(1, 64) for a in (as0, as1, as2)]
    adv = [a.reshape(1, 64) for a in (ad0, ad1, ad2)]
    bg = [b.reshape(1, 64) for b in (bg0, bg1, bg2)]

    hext, stab = _prep0(x, Wn, bn.reshape(1, 64), Wg0, asv[0], adv[0])
    for i in range(3):
        acc = _sc_call(src, dst, hext, stab[0], stab[1], stab[2])
        if i < 2:
            hext, stab = _prep_mid(acc, hext, bg[i],
                                   (Wg1, Wg2)[i], asv[i + 1], adv[i + 1])
    scores, types, ge = _head(
        acc, hext, bg[2],
        Wp1, bp1.reshape(1, 64), Wp2, bp2.reshape(1, 32),
        Wc1, bc1.reshape(1, 16), Wc2, bc2.reshape(1, 1),
        Wt1, bt1.reshape(1, 16), Wt2, bt2.reshape(1, 6))
    return (scores, types, ge)


# speedup vs baseline: 1.0667x; 1.0667x over previous
import jax
import jax.numpy as jnp
from jax import lax
from jax.experimental import pallas as pl
from jax.experimental.pallas import tpu as pltpu
from jax.experimental.pallas import tpu_sc as plsc

N = 10000
E = 320000
NEG = 0.2
FW = 64
CB = 80
NSLOT = 5
TILES = 32
EPT = E // TILES
CHUNKS = EPT // CB
NGROUP = CHUNKS // NSLOT
RPT = 632
RPT_LAST = N - 15 * RPT


def _attn_tables(act, Wg_ref, asv_ref, adv_ref, hfeat_ref, stab_ref):
    g = jnp.dot(act, Wg_ref[...], preferred_element_type=jnp.float32)
    a_s = lax.dot_general(asv_ref[...], g, (((1,), (1,)), ((), ())),
                          preferred_element_type=jnp.float32)
    a_d = lax.dot_general(adv_ref[...], g, (((1,), (1,)), ((), ())),
                          preferred_element_type=jnp.float32)
    s = a_s + a_d
    c = jnp.where(s >= 0, s, NEG * s)
    stab_ref[...] = jnp.concatenate(
        [a_s, a_d, c, jnp.zeros((5, N), jnp.float32)], axis=0)
    hfeat_ref[...] = g


def _prep0_body(x_ref, Wn_ref, bn_ref, Wg_ref, asv_ref, adv_ref,
                hfeat_ref, stab_ref):
    act = jnp.dot(x_ref[...], Wn_ref[...],
                  preferred_element_type=jnp.float32) + bn_ref[...]
    _attn_tables(act, Wg_ref, asv_ref, adv_ref, hfeat_ref, stab_ref)


def _gat_out(acc_ref, d0_ref, d1_ref, hfeatp_ref, bg_ref):
    num = acc_ref[0] + acc_ref[1] - hfeatp_ref[...]
    den = d0_ref[...] + d1_ref[...] + (1.0 + 1e-16)
    return num / den + bg_ref[...]


def _prep_mid_body(acc_ref, d0_ref, d1_ref, hfeatp_ref, bg_ref,
                   Wg_ref, asv_ref, adv_ref, hfeat_ref, stab_ref):
    act = jnp.maximum(_gat_out(acc_ref, d0_ref, d1_ref, hfeatp_ref, bg_ref),
                      0.0)
    _attn_tables(act, Wg_ref, asv_ref, adv_ref, hfeat_ref, stab_ref)


def _head_body(acc_ref, d0_ref, d1_ref, hfeatp_ref, bg_ref,
               Wp1_ref, bp1_ref, Wp2_ref, bp2_ref,
               Wc1_ref, bc1_ref, Wc2_ref, bc2_ref,
               Wt1_ref, bt1_ref, Wt2_ref, bt2_ref,
               scores_ref, types_ref, ge_ref):
    h = _gat_out(acc_ref, d0_ref, d1_ref, hfeatp_ref, bg_ref)
    gm = jnp.mean(h, axis=0, keepdims=True)
    gx = jnp.max(h, axis=0, keepdims=True)
    gr = jnp.concatenate([gm, gx], axis=1)
    ge = jnp.maximum(gr @ Wp1_ref[...] + bp1_ref[...], 0.0) @ Wp2_ref[...] \
        + bp2_ref[...]
    scores_ref[...] = jax.nn.sigmoid(
        jnp.maximum(ge @ Wc1_ref[...] + bc1_ref[...], 0.0) @ Wc2_ref[...]
        + bc2_ref[...])
    types_ref[...] = jnp.maximum(ge @ Wt1_ref[...] + bt1_ref[...], 0.0) \
        @ Wt2_ref[...] + bt2_ref[...]
    ge_ref[...] = ge


def _full16(v):
    return jnp.full((16,), v, jnp.int32)


def _sc_edge(src_hbm, dst_hbm, hfeat_hbm, as_hbm, ad_hbm, c_hbm, z_hbm,
             acc_hbm, den0_hbm, den1_hbm,
             tabA, tabB, tabC, srcidx, dstidx, rows, wbuf, acc_sh, den_sh,
             ids_sem, gat_sem, scr_sem, scd_sem):
    cid = lax.axis_index("c")
    sid = lax.axis_index("s")
    wid = cid * 16 + sid
    pltpu.sync_copy(as_hbm, tabA)
    pltpu.sync_copy(ad_hbm, tabB)
    pltpu.sync_copy(c_hbm, tabC)
    r0 = sid * RPT

    @pl.when(sid < 15)
    def _():
        pltpu.sync_copy(hfeat_hbm.at[pl.ds(r0, RPT)], acc_sh.at[pl.ds(r0, RPT)])
        pltpu.sync_copy(z_hbm.at[pl.ds(r0, RPT)], den_sh.at[pl.ds(r0, RPT)])

    @pl.when(sid == 15)
    def _():
        pltpu.sync_copy(hfeat_hbm.at[pl.ds(15 * RPT, RPT_LAST)],
                        acc_sh.at[pl.ds(15 * RPT, RPT_LAST)])
        pltpu.sync_copy(z_hbm.at[pl.ds(15 * RPT, RPT_LAST)],
                        den_sh.at[pl.ds(15 * RPT, RPT_LAST)])

    plsc.subcore_barrier()

    ebase = wid * EPT

    def _fire_ids(k, s):
        base = ebase + k * CB
        pltpu.async_copy(src_hbm.at[pl.ds(base, CB)], srcidx.at[s],
                         ids_sem.at[s])
        pltpu.async_copy(dst_hbm.at[pl.ds(base, CB)], dstidx.at[s],
                         ids_sem.at[s])

    def _wait_ids(k, s):
        base = ebase + k * CB
        pltpu.make_async_copy(src_hbm.at[pl.ds(base, CB)], srcidx.at[s],
                              ids_sem.at[s]).wait()
        pltpu.make_async_copy(dst_hbm.at[pl.ds(base, CB)], dstidx.at[s],
                              ids_sem.at[s]).wait()

    def _fire_gather(s):
        pltpu.async_copy(hfeat_hbm.at[srcidx.at[s]], rows.at[s],
                         gat_sem.at[s])

    def _wait_gather(s):
        pltpu.make_async_copy(hfeat_hbm.at[srcidx.at[s]], rows.at[s],
                              gat_sem.at[s]).wait()

    def _fire_scat(s):
        pltpu.async_copy(rows.at[s], acc_sh.at[dstidx.at[s]],
                         scr_sem.at[s], add=True)
        pltpu.async_copy(wbuf.at[s], den_sh.at[dstidx.at[s]],
                         scd_sem.at[s], add=True)

    def _wait_scat(s):
        pltpu.make_async_copy(rows.at[s], acc_sh.at[dstidx.at[s]],
                              scr_sem.at[s]).wait()
        pltpu.make_async_copy(wbuf.at[s], den_sh.at[dstidx.at[s]],
                              scd_sem.at[s]).wait()

    for s in range(3):
        _fire_ids(s, s)
    for s in range(2):
        _wait_ids(s, s)
        _fire_gather(s)

    def group(gi, carry):
        for s in range(NSLOT):
            k = gi * NSLOT + s
            _wait_gather(s)
            for g5 in range(CB // 16):
                sl = pl.ds(g5 * 16, 16)
                sidx = srcidx[s, sl]
                didx = dstidx[s, sl]
                a_s = plsc.load_gather(tabA, [sidx])
                a_d = plsc.load_gather(tabB, [didx])
                cc = plsc.load_gather(tabC, [didx])
                t = a_s + a_d
                e = jnp.where(t >= 0, t, NEG * t)
                w = jnp.exp(e - cc)
                wbuf[s, sl] = w
                for l in range(16):
                    i = g5 * 16 + l
                    wl = w.at[_full16(l)].get(mode="promise_in_bounds")
                    for jb in range(FW // 16):
                        slj = pl.ds(jb * 16, 16)
                        rows[s, i, slj] = rows[s, i, slj] * wl
            _fire_scat(s)

            s3 = (s + 3) % NSLOT

            @pl.when(k < CHUNKS - 3)
            def _():
                @pl.when(k >= 2)
                def _():
                    _wait_scat(s3)

                _fire_ids(k + 3, s3)

            s2 = (s + 2) % NSLOT

            @pl.when(k < CHUNKS - 2)
            def _():
                _wait_ids(k + 2, s2)
                _fire_gather(s2)

        return carry

    lax.fori_loop(0, NGROUP, group, 0)
    for s in range(NSLOT):
        _wait_scat(s)
    plsc.subcore_barrier()

    @pl.when(sid < 15)
    def _():
        pltpu.sync_copy(acc_sh.at[pl.ds(r0, RPT)],
                        acc_hbm.at[cid, pl.ds(r0, RPT)])

    @pl.when(sid == 15)
    def _():
        pltpu.sync_copy(acc_sh.at[pl.ds(15 * RPT, RPT_LAST)],
                        acc_hbm.at[cid, pl.ds(15 * RPT, RPT_LAST)])

    @pl.when((sid < 15) & (cid == 0))
    def _():
        pltpu.sync_copy(den_sh.at[pl.ds(r0, RPT)], den0_hbm.at[pl.ds(r0, RPT)])

    @pl.when((sid == 15) & (cid == 0))
    def _():
        pltpu.sync_copy(den_sh.at[pl.ds(15 * RPT, RPT_LAST)],
                        den0_hbm.at[pl.ds(15 * RPT, RPT_LAST)])

    @pl.when((sid < 15) & (cid == 1))
    def _():
        pltpu.sync_copy(den_sh.at[pl.ds(r0, RPT)], den1_hbm.at[pl.ds(r0, RPT)])

    @pl.when((sid == 15) & (cid == 1))
    def _():
        pltpu.sync_copy(den_sh.at[pl.ds(15 * RPT, RPT_LAST)],
                        den1_hbm.at[pl.ds(15 * RPT, RPT_LAST)])


_sc_call = pl.kernel(
    _sc_edge,
    mesh=plsc.VectorSubcoreMesh(core_axis_name="c", subcore_axis_name="s"),
    out_type=(jax.ShapeDtypeStruct((2, N, FW), jnp.float32),
              jax.ShapeDtypeStruct((N,), jnp.float32),
              jax.ShapeDtypeStruct((N,), jnp.float32)),
    scratch_types=[
        pltpu.VMEM((N,), jnp.float32),
        pltpu.VMEM((N,), jnp.float32),
        pltpu.VMEM((N,), jnp.float32),
        pltpu.VMEM((NSLOT, CB), jnp.int32),
        pltpu.VMEM((NSLOT, CB), jnp.int32),
        pltpu.VMEM((NSLOT, CB, FW), jnp.float32),
        pltpu.VMEM((NSLOT, CB), jnp.float32),
        pltpu.VMEM_SHARED((N, FW), jnp.float32),
        pltpu.VMEM_SHARED((N,), jnp.float32),
        pltpu.SemaphoreType.DMA((NSLOT,)),
        pltpu.SemaphoreType.DMA((NSLOT,)),
        pltpu.SemaphoreType.DMA((NSLOT,)),
        pltpu.SemaphoreType.DMA((NSLOT,)),
    ],
    compiler_params=pltpu.CompilerParams(needs_layout_passes=False,
                                         use_tc_tiling_on_sc=False),
)

_prep0 = pl.pallas_call(
    _prep0_body,
    out_shape=(jax.ShapeDtypeStruct((N, FW), jnp.float32),
               jax.ShapeDtypeStruct((8, N), jnp.float32)),
)

_prep_mid = pl.pallas_call(
    _prep_mid_body,
    out_shape=(jax.ShapeDtypeStruct((N, FW), jnp.float32),
               jax.ShapeDtypeStruct((8, N), jnp.float32)),
)

_head = pl.pallas_call(
    _head_body,
    out_shape=(jax.ShapeDtypeStruct((1, 1), jnp.float32),
               jax.ShapeDtypeStruct((1, 6), jnp.float32),
               jax.ShapeDtypeStruct((1, 32), jnp.float32)),
)


def kernel(x, edge_index, edge_attr, Wn, bn, Wg0, as0, ad0, bg0,
           Wg1, as1, ad1, bg1, Wg2, as2, ad2, bg2,
           Wp1, bp1, Wp2, bp2, Wc1, bc1, Wc2, bc2, Wt1, bt1, Wt2, bt2):
    src = edge_index[0]
    dst = edge_index[1]
    asv = [a.reshape(1, 64) for a in (as0, as1, as2)]
    adv = [a.reshape(1, 64) for a in (ad0, ad1, ad2)]
    bg = [b.reshape(1, 64) for b in (bg0, bg1, bg2)]

    hfeat, stab = _prep0(x, Wn, bn.reshape(1, 64), Wg0, asv[0], adv[0])
    for i in range(3):
        acc, den0, den1 = _sc_call(src, dst, hfeat,
                                   stab[0], stab[1], stab[2], stab[3])
        d0 = den0.reshape(N, 1)
        d1 = den1.reshape(N, 1)
        if i < 2:
            hfeat, stab = _prep_mid(acc, d0, d1, hfeat, bg[i],
                                    (Wg1, Wg2)[i], asv[i + 1], adv[i + 1])
    scores, types, ge = _head(
        acc, d0, d1, hfeat, bg[2],
        Wp1, bp1.reshape(1, 64), Wp2, bp2.reshape(1, 32),
        Wc1, bc1.reshape(1, 16), Wc2, bc2.reshape(1, 1),
        Wt1, bt1.reshape(1, 16), Wt2, bt2.reshape(1, 6))
    return (scores, types, ge)
